# baseline (speedup 1.0000x reference)
import functools

import jax
import jax.numpy as jnp
from jax import lax
from jax.experimental import pallas as pl
from jax.experimental.pallas import tpu as pltpu
from jax.experimental.pallas import tpu_sc as plsc

N = 4096
H = 1024
E = 8
F = 2048
D = 64
RT = 64

NC = 2
NS = 16
NW = NC * NS
TOK_W = N // NW

T = 128
NT = N // T + E
NT_PAD = 48
PMAX = NT * T

def _router_body(x_ref, wr1_ref, br1_ref, wr2_ref, br2_ref, eemb_ref,
                 gum_ref, idx_ref, bcnt_ref, lbl_ref, ent_ref):
    x = x_ref[...]
    h = jnp.maximum(
        jnp.dot(x, wr1_ref[...], preferred_element_type=jnp.float32)
        + br1_ref[...], 0.0)
    emb = (jnp.dot(h, wr2_ref[...], preferred_element_type=jnp.float32)
           + br2_ref[...])
    logits = lax.dot_general(
        emb, eemb_ref[...], (((1,), (1,)), ((), ())),
        preferred_element_type=jnp.float32)

    z = logits + gum_ref[...]
    zm = jnp.max(z, axis=-1, keepdims=True)
    ez = jnp.exp(z - zm)
    y_soft = ez / jnp.sum(ez, axis=-1, keepdims=True)
    ym = jnp.max(y_soft, axis=-1, keepdims=True)
    iota_e = lax.broadcasted_iota(jnp.int32, (N, E), 1)
    cand = jnp.where(y_soft == ym, iota_e, E)
    idx = jnp.min(cand, axis=-1)
    idx_ref[...] = idx

    lm = jnp.max(logits, axis=-1, keepdims=True)
    el = jnp.exp(logits - lm)
    probs = el / jnp.sum(el, axis=-1, keepdims=True)

    iota16 = lax.broadcasted_iota(jnp.int32, (N, 16), 1)
    oh16 = (iota16 == idx[:, None]).astype(jnp.float32)
    frac = jnp.mean(oh16[:, :E], axis=0)
    pm = jnp.mean(probs, axis=0)
    lbl_ref[...] = jnp.reshape(E * jnp.sum(frac * pm), (1, 1))
    ent_ref[...] = jnp.reshape(
        -jnp.mean(jnp.sum(probs * jnp.log(probs + 1e-9), axis=-1)), (1, 1))

    row_b = lax.broadcasted_iota(jnp.int32, (NW, N), 0)
    col_b = lax.broadcasted_iota(jnp.int32, (NW, N), 1)
    sel = (col_b // TOK_W == row_b).astype(jnp.float32)
    bcnt = jnp.dot(sel, oh16, preferred_element_type=jnp.float32)
    bcnt_ref[...] = bcnt.astype(jnp.int32)


def _router(x, wr1, br1, wr2, br2, eemb, gum):
    return pl.pallas_call(
        _router_body,
        out_shape=[
            jax.ShapeDtypeStruct((N,), jnp.int32),
            jax.ShapeDtypeStruct((NW, 16), jnp.int32),
            jax.ShapeDtypeStruct((1, 1), jnp.float32),
            jax.ShapeDtypeStruct((1, 1), jnp.float32),
        ],
    )(x, wr1, br1, wr2, br2, eemb, gum)


def _sc_dispatch_body(bcnt_hbm, idx_hbm, x_hbm, xs_hbm, pos_hbm, te_hbm,
                      cnts_v, idx_v, pos_v, xbuf_v, te_v, sem):
    wid = lax.axis_index("s") * NC + lax.axis_index("c")
    base = wid * TOK_W
    pltpu.sync_copy(bcnt_hbm, cnts_v)
    pltpu.sync_copy(idx_hbm.at[pl.ds(base, TOK_W)], idx_v)

    lane = lax.iota(jnp.int32, 16)
    tot = jnp.zeros((16,), jnp.int32)
    pre = jnp.zeros((16,), jnp.int32)
    for w in range(NW):
        row = cnts_v[w, :]
        tot = tot + row
        pre = pre + row * (jnp.int32(w) < wid).astype(jnp.int32)

    seven = jnp.full((16,), 7, jnp.int32)
    padded = lax.shift_left(lax.shift_right_logical(tot + (T - 1), seven), seven)
    inc = plsc.cumsum(padded)
    exc = inc - padded
    base_vec = exc + pre

    neg = jnp.int32(-2147483648)
    bs = [jnp.max(jnp.where(lane == e, base_vec, neg)) for e in range(E)]
    ends = [jnp.max(jnp.where(lane == e, inc, neg)) for e in range(E)]
    run = [jnp.int32(0)] * E

    for c in range(TOK_W // 16):
        v = idx_v[pl.ds(c * 16, 16)]
        posc = jnp.zeros((16,), jnp.int32)
        for e in range(E):
            m = v == e
            r = plsc.cumsum(m.astype(jnp.int32))
            posc = jnp.where(m, bs[e] + run[e] + (r - 1), posc)
            run[e] = run[e] + jnp.max(r)
        pos_v[c // 4, pl.ds((c % 4) * 16, 16)] = posc

    for hh in range(2):
        pltpu.sync_copy(pos_v.at[hh], pos_hbm.at[pl.ds(base + hh * 64, 64)])
        pltpu.sync_copy(x_hbm.at[pl.ds(base + hh * 64, 64)], xbuf_v)
        pltpu.async_copy(xbuf_v, xs_hbm.at[pos_v.at[hh]], sem).wait()

    @pl.when(wid == 0)
    def _():
        for k in range(NT_PAD // 16):
            tvec = (lax.iota(jnp.int32, 16) + k * 16) * T
            cnt = jnp.zeros((16,), jnp.int32)
            for e in range(E):
                cnt = cnt + (tvec >= ends[e]).astype(jnp.int32)
            te_v[pl.ds(k * 16, 16)] = cnt
        pltpu.sync_copy(te_v, te_hbm)


NF = 2
FC = F // NF


def _expert_body(te_ref, xs_ref, w1_ref, b1_ref, w2_ref, b2_ref, y_ref):
    t = pl.program_id(0)
    fc = pl.program_id(1)

    @pl.when(te_ref[t] < E)
    def _():
        h = jnp.maximum(
            jnp.dot(xs_ref[...], w1_ref[0],
                    preferred_element_type=jnp.float32) + b1_ref[0], 0.0)
        part = jnp.dot(h, w2_ref[0], preferred_element_type=jnp.float32)

        @pl.when(fc == 0)
        def _():
            y_ref[...] = part + b2_ref[0]

        @pl.when(fc != 0)
        def _():
            y_ref[...] += part


def _experts(te, xs, w1, b1, w2, b2):
    def emap_w1(t, fc, s):
        return (jnp.minimum(s[t], E - 1), 0, fc)

    def emap_b1(t, fc, s):
        return (jnp.minimum(s[t], E - 1), 0, fc)

    def emap_w2(t, fc, s):
        return (jnp.minimum(s[t], E - 1), fc, 0)

    def emap_b2(t, fc, s):
        return (jnp.minimum(s[t], E - 1), 0, 0)

    grid_spec = pltpu.PrefetchScalarGridSpec(
        num_scalar_prefetch=1,
        grid=(NT, NF),
        in_specs=[
            pl.BlockSpec((T, H), lambda t, fc, s: (t, 0)),
            pl.BlockSpec((1, H, FC), emap_w1),
            pl.BlockSpec((1, 1, FC), emap_b1),
            pl.BlockSpec((1, FC, H), emap_w2),
            pl.BlockSpec((1, 1, H), emap_b2),
        ],
        out_specs=pl.BlockSpec((T, H), lambda t, fc, s: (t, 0)),
    )
    return pl.pallas_call(
        _expert_body,
        grid_spec=grid_spec,
        out_shape=jax.ShapeDtypeStruct((PMAX, H), jnp.float32),
        compiler_params=pltpu.CompilerParams(
            dimension_semantics=("arbitrary", "arbitrary")),
    )(te, xs, w1, b1, w2, b2)


def _sc_unsort_body(pos_hbm, ys_hbm, out_hbm, pos_v, ybuf_v, sem):
    wid = lax.axis_index("s") * NC + lax.axis_index("c")
    base = wid * TOK_W
    for hh in range(2):
        pltpu.sync_copy(pos_hbm.at[pl.ds(base + hh * 64, 64)], pos_v)
        pltpu.async_copy(ys_hbm.at[pos_v], ybuf_v, sem).wait()
        pltpu.sync_copy(ybuf_v, out_hbm.at[pl.ds(base + hh * 64, 64)])


@functools.lru_cache(maxsize=1)
def _sc_kernels():
    mesh = plsc.VectorSubcoreMesh(
        core_axis_name="c", subcore_axis_name="s",
        num_cores=NC, num_subcores=NS)
    dispatch = pl.kernel(
        _sc_dispatch_body,
        out_type=[
            jax.ShapeDtypeStruct((PMAX, H), jnp.float32),
            jax.ShapeDtypeStruct((N,), jnp.int32),
            jax.ShapeDtypeStruct((NT_PAD,), jnp.int32),
        ],
        mesh=mesh,
        compiler_params=pltpu.CompilerParams(needs_layout_passes=False),
        scratch_types=[
            pltpu.VMEM((NW, 16), jnp.int32),
            pltpu.VMEM((TOK_W,), jnp.int32),
            pltpu.VMEM((2, 64), jnp.int32),
            pltpu.VMEM((64, H), jnp.float32),
            pltpu.VMEM((NT_PAD,), jnp.int32),
            pltpu.SemaphoreType.DMA,
        ],
    )
    unsort = pl.kernel(
        _sc_unsort_body,
        out_type=jax.ShapeDtypeStruct((N, H), jnp.float32),
        mesh=mesh,
        compiler_params=pltpu.CompilerParams(needs_layout_passes=False),
        scratch_types=[
            pltpu.VMEM((64,), jnp.int32),
            pltpu.VMEM((64, H), jnp.float32),
            pltpu.SemaphoreType.DMA,
        ],
    )
    return dispatch, unsort


def kernel(x, W1, b1, W2, b2, Wr1, br1, Wr2, br2, Eemb):
    gkey = jax.random.key(42)
    u = jax.random.uniform(gkey, (N, E), dtype=jnp.float32,
                           minval=1e-10, maxval=1.0)
    gumbel = -jnp.log(-jnp.log(u))

    idx, bcnt, lbl, ent = _router(
        x, Wr1, br1.reshape(1, RT), Wr2, br2.reshape(1, D), Eemb, gumbel)
    _sc_dispatch, _sc_unsort = _sc_kernels()
    xs, pos, te = _sc_dispatch(bcnt, idx, x)
    ys = _experts(te, xs, W1, b1.reshape(E, 1, F), W2, b2.reshape(E, 1, H))
    out = _sc_unsort(pos, ys)
    return out, jnp.reshape(lbl, ()), jnp.reshape(ent, ())


# speedup vs baseline: 1.8396x; 1.8396x over previous
import functools

import numpy as np

import jax
import jax.numpy as jnp
from jax import lax
from jax.experimental import pallas as pl
from jax.experimental.pallas import tpu as pltpu
from jax.experimental.pallas import tpu_sc as plsc

N = 4096
H = 1024
E = 8
F = 2048
D = 64
RT = 64

NC = 2
NS = 16
NW = NC * NS
TOK_W = N // NW

T = 128
NT = N // T + E
NT_PAD = 48
PMAX = NT * T

def _threefry2x32_np(k0, k1, x0, x1):
    rot = [(13, 15, 26, 6), (17, 29, 16, 24)]
    ks = [np.uint32(k0), np.uint32(k1), np.uint32(k0 ^ k1 ^ 0x1BD11BDA)]
    x = [(x0 + ks[0]).astype(np.uint32), (x1 + ks[1]).astype(np.uint32)]

    def rotl(v, d):
        return ((v << np.uint32(d)) | (v >> np.uint32(32 - d))).astype(np.uint32)

    for i in range(5):
        for r in rot[i % 2]:
            x[0] = (x[0] + x[1]).astype(np.uint32)
            x[1] = rotl(x[1], r)
            x[1] = x[1] ^ x[0]
        x[0] = (x[0] + ks[(i + 1) % 3]).astype(np.uint32)
        x[1] = (x[1] + ks[(i + 2) % 3] + np.uint32(i + 1)).astype(np.uint32)
    return x


def _gumbel_t_np():
    n = N * E
    o0, o1 = _threefry2x32_np(0, 42, np.zeros(n, np.uint32),
                              np.arange(n, dtype=np.uint32))
    bits = (o0 ^ o1).astype(np.uint32)
    fb = (bits >> np.uint32(9)) | np.uint32(0x3F800000)
    fl = fb.view(np.float32) - np.float32(1.0)
    lo = np.float32(1e-10)
    u = np.maximum(lo, fl * (np.float32(1.0) - lo) + lo).reshape(N, E)
    return np.ascontiguousarray((-np.log(-np.log(u))).T)


_GUM_T = _gumbel_t_np()


def _router_body(x_ref, wr1_ref, br1_ref, wr2_ref, br2_ref, eemb_ref,
                 gum_ref, idx_ref, bcnt_ref, lbl_ref, ent_ref):
    x = x_ref[...]
    h = jnp.maximum(
        jnp.dot(x, wr1_ref[...], preferred_element_type=jnp.float32)
        + br1_ref[...], 0.0)
    emb = (jnp.dot(h, wr2_ref[...], preferred_element_type=jnp.float32)
           + br2_ref[...])
    logits_t = lax.dot_general(
        eemb_ref[...], emb, (((1,), (1,)), ((), ())),
        preferred_element_type=jnp.float32)

    z = logits_t + gum_ref[...]
    zm = jnp.max(z, axis=0, keepdims=True)
    ez = jnp.exp(z - zm)
    y_soft = ez / jnp.sum(ez, axis=0, keepdims=True)
    ym = jnp.max(y_soft, axis=0, keepdims=True)
    iota_e = lax.broadcasted_iota(jnp.int32, (E, N), 0)
    cand = jnp.where(y_soft == ym, iota_e, E)
    idx = jnp.min(cand, axis=0)
    idx_ref[...] = idx

    lm = jnp.max(logits_t, axis=0, keepdims=True)
    el = jnp.exp(logits_t - lm)
    probs = el / jnp.sum(el, axis=0, keepdims=True)

    iota16 = lax.broadcasted_iota(jnp.int32, (16, N), 0)
    oh16 = (iota16 == idx[None, :]).astype(jnp.float32)
    frac = jnp.mean(oh16[:E, :], axis=1)
    pm = jnp.mean(probs, axis=1)
    lbl_ref[...] = jnp.reshape(E * jnp.sum(frac * pm), (1, 1))
    ent_ref[...] = jnp.reshape(
        -jnp.mean(jnp.sum(probs * jnp.log(probs + 1e-9), axis=0)), (1, 1))

    row_b = lax.broadcasted_iota(jnp.int32, (NW, N), 0)
    col_b = lax.broadcasted_iota(jnp.int32, (NW, N), 1)
    sel = (jnp.right_shift(col_b, 7) == row_b).astype(jnp.float32)
    bcnt = lax.dot_general(sel, oh16, (((1,), (1,)), ((), ())),
                           preferred_element_type=jnp.float32)
    bcnt_ref[...] = bcnt.astype(jnp.int32)


def _router(x, wr1, br1, wr2, br2, eemb, gum_t):
    return pl.pallas_call(
        _router_body,
        out_shape=[
            jax.ShapeDtypeStruct((N,), jnp.int32),
            jax.ShapeDtypeStruct((NW, 16), jnp.int32),
            jax.ShapeDtypeStruct((1, 1), jnp.float32),
            jax.ShapeDtypeStruct((1, 1), jnp.float32),
        ],
    )(x, wr1, br1, wr2, br2, eemb, gum_t)


def _sc_dispatch_body(bcnt_hbm, idx_hbm, x_hbm, xs_hbm, pos_hbm, te_hbm,
                      cnts_v, idx_v, pos_v, xbuf_v, te_v, sem):
    wid = lax.axis_index("s") * NC + lax.axis_index("c")
    base = wid * TOK_W
    pltpu.sync_copy(bcnt_hbm, cnts_v)
    pltpu.sync_copy(idx_hbm.at[pl.ds(base, TOK_W)], idx_v)

    lane = lax.iota(jnp.int32, 16)
    tot = jnp.zeros((16,), jnp.int32)
    pre = jnp.zeros((16,), jnp.int32)
    for w in range(NW):
        row = cnts_v[w, :]
        tot = tot + row
        pre = pre + row * (jnp.int32(w) < wid).astype(jnp.int32)

    seven = jnp.full((16,), 7, jnp.int32)
    padded = lax.shift_left(lax.shift_right_logical(tot + (T - 1), seven), seven)
    inc = plsc.cumsum(padded)
    exc = inc - padded
    base_vec = exc + pre

    neg = jnp.int32(-2147483648)
    bs = [jnp.max(jnp.where(lane == e, base_vec, neg)) for e in range(E)]
    ends = [jnp.max(jnp.where(lane == e, inc, neg)) for e in range(E)]
    run = [jnp.int32(0)] * E

    for c in range(TOK_W // 16):
        v = idx_v[pl.ds(c * 16, 16)]
        posc = jnp.zeros((16,), jnp.int32)
        for e in range(E):
            m = v == e
            r = plsc.cumsum(m.astype(jnp.int32))
            posc = jnp.where(m, bs[e] + run[e] + (r - 1), posc)
            run[e] = run[e] + jnp.max(r)
        pos_v[c // 4, pl.ds((c % 4) * 16, 16)] = posc

    for hh in range(2):
        pltpu.sync_copy(pos_v.at[hh], pos_hbm.at[pl.ds(base + hh * 64, 64)])
        pltpu.sync_copy(x_hbm.at[pl.ds(base + hh * 64, 64)], xbuf_v)
        pltpu.async_copy(xbuf_v, xs_hbm.at[pos_v.at[hh]], sem).wait()

    @pl.when(wid == 0)
    def _():
        for k in range(NT_PAD // 16):
            tvec = (lax.iota(jnp.int32, 16) + k * 16) * T
            cnt = jnp.zeros((16,), jnp.int32)
            for e in range(E):
                cnt = cnt + (tvec >= ends[e]).astype(jnp.int32)
            te_v[pl.ds(k * 16, 16)] = cnt
        pltpu.sync_copy(te_v, te_hbm)


def _expert_body(te_ref, xs_ref, w1_ref, b1_ref, w2_ref, b2_ref, y_ref):
    t = pl.program_id(0)

    @pl.when(te_ref[t] < E)
    def _():
        h = jnp.maximum(
            jnp.dot(xs_ref[...], w1_ref[0],
                    preferred_element_type=jnp.float32) + b1_ref[0], 0.0)
        y_ref[...] = (jnp.dot(h, w2_ref[0],
                              preferred_element_type=jnp.float32)
                      + b2_ref[0])


def _experts(te, xs, w1, b1, w2, b2):
    def emap(t, s):
        return (jnp.minimum(s[t], E - 1), 0, 0)

    grid_spec = pltpu.PrefetchScalarGridSpec(
        num_scalar_prefetch=1,
        grid=(NT,),
        in_specs=[
            pl.BlockSpec((T, H), lambda t, s: (t, 0)),
            pl.BlockSpec((1, H, F), emap),
            pl.BlockSpec((1, 1, F), emap),
            pl.BlockSpec((1, F, H), emap),
            pl.BlockSpec((1, 1, H), emap),
        ],
        out_specs=pl.BlockSpec((T, H), lambda t, s: (t, 0)),
    )
    return pl.pallas_call(
        _expert_body,
        grid_spec=grid_spec,
        out_shape=jax.ShapeDtypeStruct((PMAX, H), jnp.float32),
        compiler_params=pltpu.CompilerParams(
            dimension_semantics=("arbitrary",)),
    )(te, xs, w1, b1, w2, b2)


def _sc_unsort_body(pos_hbm, ys_hbm, out_hbm, pos_v, ybuf_v, sem):
    wid = lax.axis_index("s") * NC + lax.axis_index("c")
    base = wid * TOK_W
    for hh in range(2):
        pltpu.sync_copy(pos_hbm.at[pl.ds(base + hh * 64, 64)], pos_v)
        pltpu.async_copy(ys_hbm.at[pos_v], ybuf_v, sem).wait()
        pltpu.sync_copy(ybuf_v, out_hbm.at[pl.ds(base + hh * 64, 64)])


@functools.lru_cache(maxsize=1)
def _sc_kernels():
    mesh = plsc.VectorSubcoreMesh(
        core_axis_name="c", subcore_axis_name="s",
        num_cores=NC, num_subcores=NS)
    dispatch = pl.kernel(
        _sc_dispatch_body,
        out_type=[
            jax.ShapeDtypeStruct((PMAX, H), jnp.float32),
            jax.ShapeDtypeStruct((N,), jnp.int32),
            jax.ShapeDtypeStruct((NT_PAD,), jnp.int32),
        ],
        mesh=mesh,
        compiler_params=pltpu.CompilerParams(needs_layout_passes=False),
        scratch_types=[
            pltpu.VMEM((NW, 16), jnp.int32),
            pltpu.VMEM((TOK_W,), jnp.int32),
            pltpu.VMEM((2, 64), jnp.int32),
            pltpu.VMEM((64, H), jnp.float32),
            pltpu.VMEM((NT_PAD,), jnp.int32),
            pltpu.SemaphoreType.DMA,
        ],
    )
    unsort = pl.kernel(
        _sc_unsort_body,
        out_type=jax.ShapeDtypeStruct((N, H), jnp.float32),
        mesh=mesh,
        compiler_params=pltpu.CompilerParams(needs_layout_passes=False),
        scratch_types=[
            pltpu.VMEM((64,), jnp.int32),
            pltpu.VMEM((64, H), jnp.float32),
            pltpu.SemaphoreType.DMA,
        ],
    )
    return dispatch, unsort


def kernel(x, W1, b1, W2, b2, Wr1, br1, Wr2, br2, Eemb):
    idx, bcnt, lbl, ent = _router(
        x, Wr1, br1.reshape(1, RT), Wr2, br2.reshape(1, D), Eemb,
        jnp.asarray(_GUM_T))
    _sc_dispatch, _sc_unsort = _sc_kernels()
    xs, pos, te = _sc_dispatch(bcnt, idx, x)
    ys = _experts(te, xs, W1, b1.reshape(E, 1, F), W2, b2.reshape(E, 1, H))
    out = _sc_unsort(pos, ys)
    return out, jnp.reshape(lbl, ()), jnp.reshape(ent, ())


# speedup vs baseline: 2.0594x; 1.1195x over previous
import functools

import numpy as np

import jax
import jax.numpy as jnp
from jax import lax
from jax.experimental import pallas as pl
from jax.experimental.pallas import tpu as pltpu
from jax.experimental.pallas import tpu_sc as plsc

N = 4096
H = 1024
E = 8
F = 2048
D = 64
RT = 64

NC = 2
NS = 16
NW = NC * NS
TOK_W = N // NW

T = 512
TSH = 9
NT = N // T + E
NT_PAD = 48
PMAX = NT * T

def _threefry2x32_np(k0, k1, x0, x1):
    rot = [(13, 15, 26, 6), (17, 29, 16, 24)]
    ks = [np.uint32(k0), np.uint32(k1), np.uint32(k0 ^ k1 ^ 0x1BD11BDA)]
    x = [(x0 + ks[0]).astype(np.uint32), (x1 + ks[1]).astype(np.uint32)]

    def rotl(v, d):
        return ((v << np.uint32(d)) | (v >> np.uint32(32 - d))).astype(np.uint32)

    for i in range(5):
        for r in rot[i % 2]:
            x[0] = (x[0] + x[1]).astype(np.uint32)
            x[1] = rotl(x[1], r)
            x[1] = x[1] ^ x[0]
        x[0] = (x[0] + ks[(i + 1) % 3]).astype(np.uint32)
        x[1] = (x[1] + ks[(i + 2) % 3] + np.uint32(i + 1)).astype(np.uint32)
    return x


def _gumbel_t_np():
    n = N * E
    o0, o1 = _threefry2x32_np(0, 42, np.zeros(n, np.uint32),
                              np.arange(n, dtype=np.uint32))
    bits = (o0 ^ o1).astype(np.uint32)
    fb = (bits >> np.uint32(9)) | np.uint32(0x3F800000)
    fl = fb.view(np.float32) - np.float32(1.0)
    lo = np.float32(1e-10)
    u = np.maximum(lo, fl * (np.float32(1.0) - lo) + lo).reshape(N, E)
    return np.ascontiguousarray((-np.log(-np.log(u))).T)


_GUM_T = _gumbel_t_np()


def _router_body(x_ref, wr1_ref, br1_ref, wr2_ref, br2_ref, eemb_ref,
                 gum_ref, idx_ref, bcnt_ref, lbl_ref, ent_ref):
    x = x_ref[...]
    h = jnp.maximum(
        jnp.dot(x, wr1_ref[...], preferred_element_type=jnp.float32)
        + br1_ref[...], 0.0)
    emb = (jnp.dot(h, wr2_ref[...], preferred_element_type=jnp.float32)
           + br2_ref[...])
    logits_t = lax.dot_general(
        eemb_ref[...], emb, (((1,), (1,)), ((), ())),
        preferred_element_type=jnp.float32)

    z = logits_t + gum_ref[...]
    zm = jnp.max(z, axis=0, keepdims=True)
    ez = jnp.exp(z - zm)
    y_soft = ez / jnp.sum(ez, axis=0, keepdims=True)
    ym = jnp.max(y_soft, axis=0, keepdims=True)
    iota_e = lax.broadcasted_iota(jnp.int32, (E, N), 0)
    cand = jnp.where(y_soft == ym, iota_e, E)
    idx = jnp.min(cand, axis=0)
    idx_ref[...] = idx

    lm = jnp.max(logits_t, axis=0, keepdims=True)
    el = jnp.exp(logits_t - lm)
    probs = el / jnp.sum(el, axis=0, keepdims=True)

    iota16 = lax.broadcasted_iota(jnp.int32, (16, N), 0)
    oh16 = (iota16 == idx[None, :]).astype(jnp.float32)
    frac = jnp.mean(oh16[:E, :], axis=1)
    pm = jnp.mean(probs, axis=1)
    lbl_ref[...] = jnp.reshape(E * jnp.sum(frac * pm), (1, 1))
    ent_ref[...] = jnp.reshape(
        -jnp.mean(jnp.sum(probs * jnp.log(probs + 1e-9), axis=0)), (1, 1))

    row_b = lax.broadcasted_iota(jnp.int32, (NW, N), 0)
    col_b = lax.broadcasted_iota(jnp.int32, (NW, N), 1)
    sel = (jnp.right_shift(col_b, 7) == row_b).astype(jnp.float32)
    bcnt = lax.dot_general(sel, oh16, (((1,), (1,)), ((), ())),
                           preferred_element_type=jnp.float32)
    bcnt_ref[...] = bcnt.astype(jnp.int32)


def _router(x, wr1, br1, wr2, br2, eemb, gum_t):
    return pl.pallas_call(
        _router_body,
        out_shape=[
            jax.ShapeDtypeStruct((N,), jnp.int32),
            jax.ShapeDtypeStruct((NW, 16), jnp.int32),
            jax.ShapeDtypeStruct((1, 1), jnp.float32),
            jax.ShapeDtypeStruct((1, 1), jnp.float32),
        ],
    )(x, wr1, br1, wr2, br2, eemb, gum_t)


def _sc_dispatch_body(bcnt_hbm, idx_hbm, x_hbm, xs_hbm, pos_hbm, te_hbm,
                      cnts_v, idx_v, pos_v, xbuf_v, te_v, sem):
    wid = lax.axis_index("s") * NC + lax.axis_index("c")
    base = wid * TOK_W
    pltpu.sync_copy(bcnt_hbm, cnts_v)
    pltpu.sync_copy(idx_hbm.at[pl.ds(base, TOK_W)], idx_v)

    lane = lax.iota(jnp.int32, 16)
    tot = jnp.zeros((16,), jnp.int32)
    pre = jnp.zeros((16,), jnp.int32)
    for w in range(NW):
        row = cnts_v[w, :]
        tot = tot + row
        pre = pre + row * (jnp.int32(w) < wid).astype(jnp.int32)

    tsh = jnp.full((16,), TSH, jnp.int32)
    padded = lax.shift_left(lax.shift_right_logical(tot + (T - 1), tsh), tsh)
    inc = plsc.cumsum(padded)
    exc = inc - padded
    base_vec = exc + pre

    neg = jnp.int32(-2147483648)
    bs = [jnp.max(jnp.where(lane == e, base_vec, neg)) for e in range(E)]
    ends = [jnp.max(jnp.where(lane == e, inc, neg)) for e in range(E)]
    run = [jnp.int32(0)] * E

    for c in range(TOK_W // 16):
        v = idx_v[pl.ds(c * 16, 16)]
        posc = jnp.zeros((16,), jnp.int32)
        for e in range(E):
            m = v == e
            r = plsc.cumsum(m.astype(jnp.int32))
            posc = jnp.where(m, bs[e] + run[e] + (r - 1), posc)
            run[e] = run[e] + jnp.max(r)
        pos_v[c // 4, pl.ds((c % 4) * 16, 16)] = posc

    for hh in range(2):
        pltpu.sync_copy(pos_v.at[hh], pos_hbm.at[pl.ds(base + hh * 64, 64)])
        pltpu.sync_copy(x_hbm.at[pl.ds(base + hh * 64, 64)], xbuf_v)
        pltpu.async_copy(xbuf_v, xs_hbm.at[pos_v.at[hh]], sem).wait()

    @pl.when(wid == 0)
    def _():
        for k in range(NT_PAD // 16):
            tvec = (lax.iota(jnp.int32, 16) + k * 16) * T
            cnt = jnp.zeros((16,), jnp.int32)
            for e in range(E):
                cnt = cnt + (tvec >= ends[e]).astype(jnp.int32)
            te_v[pl.ds(k * 16, 16)] = cnt
        pltpu.sync_copy(te_v, te_hbm)


def _expert_body(te_ref, xs_ref, w1_ref, b1_ref, w2_ref, b2_ref, y_ref):
    t = pl.program_id(0)

    @pl.when(te_ref[t] < E)
    def _():
        h = jnp.maximum(
            jnp.dot(xs_ref[...], w1_ref[0],
                    preferred_element_type=jnp.float32) + b1_ref[0], 0.0)
        y_ref[...] = (jnp.dot(h, w2_ref[0],
                              preferred_element_type=jnp.float32)
                      + b2_ref[0])


def _experts(te, xs, w1, b1, w2, b2):
    def emap(t, s):
        return (jnp.minimum(s[t], E - 1), 0, 0)

    grid_spec = pltpu.PrefetchScalarGridSpec(
        num_scalar_prefetch=1,
        grid=(NT,),
        in_specs=[
            pl.BlockSpec((T, H), lambda t, s: (t, 0)),
            pl.BlockSpec((1, H, F), emap),
            pl.BlockSpec((1, 1, F), emap),
            pl.BlockSpec((1, F, H), emap),
            pl.BlockSpec((1, 1, H), emap),
        ],
        out_specs=pl.BlockSpec((T, H), lambda t, s: (t, 0)),
    )
    return pl.pallas_call(
        _expert_body,
        grid_spec=grid_spec,
        out_shape=jax.ShapeDtypeStruct((PMAX, H), jnp.float32),
        compiler_params=pltpu.CompilerParams(
            dimension_semantics=("arbitrary",)),
    )(te, xs, w1, b1, w2, b2)


def _sc_unsort_body(pos_hbm, ys_hbm, out_hbm, pos_v, ybuf_v, sem):
    wid = lax.axis_index("s") * NC + lax.axis_index("c")
    base = wid * TOK_W
    for hh in range(2):
        pltpu.sync_copy(pos_hbm.at[pl.ds(base + hh * 64, 64)], pos_v)
        pltpu.async_copy(ys_hbm.at[pos_v], ybuf_v, sem).wait()
        pltpu.sync_copy(ybuf_v, out_hbm.at[pl.ds(base + hh * 64, 64)])


@functools.lru_cache(maxsize=1)
def _sc_kernels():
    mesh = plsc.VectorSubcoreMesh(
        core_axis_name="c", subcore_axis_name="s",
        num_cores=NC, num_subcores=NS)
    dispatch = pl.kernel(
        _sc_dispatch_body,
        out_type=[
            jax.ShapeDtypeStruct((PMAX, H), jnp.float32),
            jax.ShapeDtypeStruct((N,), jnp.int32),
            jax.ShapeDtypeStruct((NT_PAD,), jnp.int32),
        ],
        mesh=mesh,
        compiler_params=pltpu.CompilerParams(needs_layout_passes=False),
        scratch_types=[
            pltpu.VMEM((NW, 16), jnp.int32),
            pltpu.VMEM((TOK_W,), jnp.int32),
            pltpu.VMEM((2, 64), jnp.int32),
            pltpu.VMEM((64, H), jnp.float32),
            pltpu.VMEM((NT_PAD,), jnp.int32),
            pltpu.SemaphoreType.DMA,
        ],
    )
    unsort = pl.kernel(
        _sc_unsort_body,
        out_type=jax.ShapeDtypeStruct((N, H), jnp.float32),
        mesh=mesh,
        compiler_params=pltpu.CompilerParams(needs_layout_passes=False),
        scratch_types=[
            pltpu.VMEM((64,), jnp.int32),
            pltpu.VMEM((64, H), jnp.float32),
            pltpu.SemaphoreType.DMA,
        ],
    )
    return dispatch, unsort


def kernel(x, W1, b1, W2, b2, Wr1, br1, Wr2, br2, Eemb):
    idx, bcnt, lbl, ent = _router(
        x, Wr1, br1.reshape(1, RT), Wr2, br2.reshape(1, D), Eemb,
        jnp.asarray(_GUM_T))
    _sc_dispatch, _sc_unsort = _sc_kernels()
    xs, pos, te = _sc_dispatch(bcnt, idx, x)
    ys = _experts(te, xs, W1, b1.reshape(E, 1, F), W2, b2.reshape(E, 1, H))
    out = _sc_unsort(pos, ys)
    return out, jnp.reshape(lbl, ()), jnp.reshape(ent, ())


# speedup vs baseline: 2.1427x; 1.0405x over previous
import functools

import numpy as np

import jax
import jax.numpy as jnp
from jax import lax
from jax.experimental import pallas as pl
from jax.experimental.pallas import tpu as pltpu
from jax.experimental.pallas import tpu_sc as plsc

N = 4096
H = 1024
E = 8
F = 2048
D = 64
RT = 64

NC = 2
NS = 16
NW = NC * NS
TOK_W = N // NW

T = 512
TSH = 9
NT = N // T + E
NT_PAD = 48
PMAX = NT * T

def _threefry2x32_np(k0, k1, x0, x1):
    rot = [(13, 15, 26, 6), (17, 29, 16, 24)]
    ks = [np.uint32(k0), np.uint32(k1), np.uint32(k0 ^ k1 ^ 0x1BD11BDA)]
    x = [(x0 + ks[0]).astype(np.uint32), (x1 + ks[1]).astype(np.uint32)]

    def rotl(v, d):
        return ((v << np.uint32(d)) | (v >> np.uint32(32 - d))).astype(np.uint32)

    for i in range(5):
        for r in rot[i % 2]:
            x[0] = (x[0] + x[1]).astype(np.uint32)
            x[1] = rotl(x[1], r)
            x[1] = x[1] ^ x[0]
        x[0] = (x[0] + ks[(i + 1) % 3]).astype(np.uint32)
        x[1] = (x[1] + ks[(i + 2) % 3] + np.uint32(i + 1)).astype(np.uint32)
    return x


def _gumbel_t_np():
    n = N * E
    o0, o1 = _threefry2x32_np(0, 42, np.zeros(n, np.uint32),
                              np.arange(n, dtype=np.uint32))
    bits = (o0 ^ o1).astype(np.uint32)
    fb = (bits >> np.uint32(9)) | np.uint32(0x3F800000)
    fl = fb.view(np.float32) - np.float32(1.0)
    lo = np.float32(1e-10)
    u = np.maximum(lo, fl * (np.float32(1.0) - lo) + lo).reshape(N, E)
    return np.ascontiguousarray((-np.log(-np.log(u))).T)


_GUM_T = _gumbel_t_np()


def _router_body(x_ref, wr1_ref, br1_ref, wr2_ref, br2_ref, eemb_ref,
                 gum_ref, idx_ref, bcnt_ref, lbl_ref, ent_ref):
    x = x_ref[...]
    h = jnp.maximum(
        jnp.dot(x, wr1_ref[...], preferred_element_type=jnp.float32)
        + br1_ref[...], 0.0)
    emb = (jnp.dot(h, wr2_ref[...], preferred_element_type=jnp.float32)
           + br2_ref[...])
    logits_t = lax.dot_general(
        eemb_ref[...], emb, (((1,), (1,)), ((), ())),
        preferred_element_type=jnp.float32)

    z = logits_t + gum_ref[...]
    zm = jnp.max(z, axis=0, keepdims=True)
    ez = jnp.exp(z - zm)
    y_soft = ez / jnp.sum(ez, axis=0, keepdims=True)
    ym = jnp.max(y_soft, axis=0, keepdims=True)
    iota_e = lax.broadcasted_iota(jnp.int32, (E, N), 0)
    cand = jnp.where(y_soft == ym, iota_e, E)
    idx = jnp.min(cand, axis=0)
    idx_ref[...] = idx

    lm = jnp.max(logits_t, axis=0, keepdims=True)
    el = jnp.exp(logits_t - lm)
    probs = el / jnp.sum(el, axis=0, keepdims=True)

    iota16 = lax.broadcasted_iota(jnp.int32, (16, N), 0)
    oh16 = (iota16 == idx[None, :]).astype(jnp.float32)
    frac = jnp.mean(oh16[:E, :], axis=1)
    pm = jnp.mean(probs, axis=1)
    lbl_ref[...] = jnp.reshape(E * jnp.sum(frac * pm), (1, 1))
    ent_ref[...] = jnp.reshape(
        -jnp.mean(jnp.sum(probs * jnp.log(probs + 1e-9), axis=0)), (1, 1))

    row_b = lax.broadcasted_iota(jnp.int32, (NW, N), 0)
    col_b = lax.broadcasted_iota(jnp.int32, (NW, N), 1)
    sel = (jnp.right_shift(col_b, 7) == row_b).astype(jnp.float32)
    bcnt = lax.dot_general(sel, oh16, (((1,), (1,)), ((), ())),
                           preferred_element_type=jnp.float32)
    bcnt_ref[...] = bcnt.astype(jnp.int32)


def _router(x, wr1, br1, wr2, br2, eemb, gum_t):
    return pl.pallas_call(
        _router_body,
        out_shape=[
            jax.ShapeDtypeStruct((N,), jnp.int32),
            jax.ShapeDtypeStruct((NW, 16), jnp.int32),
            jax.ShapeDtypeStruct((1, 1), jnp.float32),
            jax.ShapeDtypeStruct((1, 1), jnp.float32),
        ],
    )(x, wr1, br1, wr2, br2, eemb, gum_t)


def _sc_dispatch_body(bcnt_hbm, idx_hbm, x_hbm, xs_hbm, pos_hbm, te_hbm,
                      cnts_v, idx_v, pos_v, xbuf_v, te_v, sem):
    wid = lax.axis_index("s") * NC + lax.axis_index("c")
    base = wid * TOK_W
    pltpu.sync_copy(bcnt_hbm, cnts_v)
    pltpu.sync_copy(idx_hbm.at[pl.ds(base, TOK_W)], idx_v)

    lane = lax.iota(jnp.int32, 16)
    tot = jnp.zeros((16,), jnp.int32)
    pre = jnp.zeros((16,), jnp.int32)
    for w in range(NW):
        row = cnts_v[w, :]
        tot = tot + row
        pre = pre + row * (jnp.int32(w) < wid).astype(jnp.int32)

    tsh = jnp.full((16,), TSH, jnp.int32)
    padded = lax.shift_left(lax.shift_right_logical(tot + (T - 1), tsh), tsh)
    inc = plsc.cumsum(padded)
    exc = inc - padded
    base_vec = exc + pre

    neg = jnp.int32(-2147483648)
    bs = [jnp.max(jnp.where(lane == e, base_vec, neg)) for e in range(E)]
    ends = [jnp.max(jnp.where(lane == e, inc, neg)) for e in range(E)]
    run = [jnp.int32(0)] * E

    for c in range(TOK_W // 16):
        v = idx_v[pl.ds(c * 16, 16)]
        posc = jnp.zeros((16,), jnp.int32)
        for e in range(E):
            m = v == e
            r = plsc.cumsum(m.astype(jnp.int32))
            posc = jnp.where(m, bs[e] + run[e] + (r - 1), posc)
            run[e] = run[e] + jnp.max(r)
        pos_v[c // 4, pl.ds((c % 4) * 16, 16)] = posc

    for hh in range(2):
        pltpu.sync_copy(pos_v.at[hh], pos_hbm.at[pl.ds(base + hh * 64, 64)])
        pltpu.sync_copy(x_hbm.at[pl.ds(base + hh * 64, 64)], xbuf_v)
        pltpu.async_copy(xbuf_v, xs_hbm.at[pos_v.at[hh]], sem).wait()

    @pl.when(wid == 0)
    def _():
        for k in range(NT_PAD // 16):
            tvec = (lax.iota(jnp.int32, 16) + k * 16) * T
            cnt = jnp.zeros((16,), jnp.int32)
            for e in range(E):
                cnt = cnt + (tvec >= ends[e]).astype(jnp.int32)
            te_v[pl.ds(k * 16, 16)] = cnt
        pltpu.sync_copy(te_v, te_hbm)


def _expert_body(te_ref, xs_ref, w1_ref, b1_ref, w2_ref, b2_ref, y_ref):
    t = pl.program_id(0)

    @pl.when(te_ref[t] < E)
    def _():
        h = jnp.maximum(
            jnp.dot(xs_ref[...], w1_ref[0],
                    preferred_element_type=jnp.float32) + b1_ref[0], 0.0)
        y_ref[...] = (jnp.dot(h, w2_ref[0],
                              preferred_element_type=jnp.float32)
                      + b2_ref[0])


def _experts(te, xs, w1, b1, w2, b2):
    def emap(t, s):
        return (jnp.minimum(s[t], E - 1), 0, 0)

    grid_spec = pltpu.PrefetchScalarGridSpec(
        num_scalar_prefetch=1,
        grid=(NT,),
        in_specs=[
            pl.BlockSpec((T, H),
                         lambda t, s: (jnp.where(s[t] < E, t, 0), 0)),
            pl.BlockSpec((1, H, F), emap),
            pl.BlockSpec((1, 1, F), emap),
            pl.BlockSpec((1, F, H), emap),
            pl.BlockSpec((1, 1, H), emap),
        ],
        out_specs=pl.BlockSpec(
            (T, H), lambda t, s: (jnp.where(s[t] < E, t, NT - 1), 0)),
    )
    return pl.pallas_call(
        _expert_body,
        grid_spec=grid_spec,
        out_shape=jax.ShapeDtypeStruct((PMAX, H), jnp.float32),
        compiler_params=pltpu.CompilerParams(
            dimension_semantics=("arbitrary",)),
    )(te, xs, w1, b1, w2, b2)


def _sc_unsort_body(pos_hbm, ys_hbm, out_hbm, pos_v, ybuf_v, sem):
    wid = lax.axis_index("s") * NC + lax.axis_index("c")
    base = wid * TOK_W
    for hh in range(2):
        pltpu.sync_copy(pos_hbm.at[pl.ds(base + hh * 64, 64)], pos_v)
        pltpu.async_copy(ys_hbm.at[pos_v], ybuf_v, sem).wait()
        pltpu.sync_copy(ybuf_v, out_hbm.at[pl.ds(base + hh * 64, 64)])


@functools.lru_cache(maxsize=1)
def _sc_kernels():
    mesh = plsc.VectorSubcoreMesh(
        core_axis_name="c", subcore_axis_name="s",
        num_cores=NC, num_subcores=NS)
    dispatch = pl.kernel(
        _sc_dispatch_body,
        out_type=[
            jax.ShapeDtypeStruct((PMAX, H), jnp.float32),
            jax.ShapeDtypeStruct((N,), jnp.int32),
            jax.ShapeDtypeStruct((NT_PAD,), jnp.int32),
        ],
        mesh=mesh,
        compiler_params=pltpu.CompilerParams(needs_layout_passes=False),
        scratch_types=[
            pltpu.VMEM((NW, 16), jnp.int32),
            pltpu.VMEM((TOK_W,), jnp.int32),
            pltpu.VMEM((2, 64), jnp.int32),
            pltpu.VMEM((64, H), jnp.float32),
            pltpu.VMEM((NT_PAD,), jnp.int32),
            pltpu.SemaphoreType.DMA,
        ],
    )
    unsort = pl.kernel(
        _sc_unsort_body,
        out_type=jax.ShapeDtypeStruct((N, H), jnp.float32),
        mesh=mesh,
        compiler_params=pltpu.CompilerParams(needs_layout_passes=False),
        scratch_types=[
            pltpu.VMEM((64,), jnp.int32),
            pltpu.VMEM((64, H), jnp.float32),
            pltpu.SemaphoreType.DMA,
        ],
    )
    return dispatch, unsort


def kernel(x, W1, b1, W2, b2, Wr1, br1, Wr2, br2, Eemb):
    idx, bcnt, lbl, ent = _router(
        x, Wr1, br1.reshape(1, RT), Wr2, br2.reshape(1, D), Eemb,
        jnp.asarray(_GUM_T))
    _sc_dispatch, _sc_unsort = _sc_kernels()
    xs, pos, te = _sc_dispatch(bcnt, idx, x)
    ys = _experts(te, xs, W1, b1.reshape(E, 1, F), W2, b2.reshape(E, 1, H))
    out = _sc_unsort(pos, ys)
    return out, jnp.reshape(lbl, ()), jnp.reshape(ent, ())
